# Initial kernel scaffold; baseline (speedup 1.0000x reference)
import jax
import jax.numpy as jnp
from jax.experimental import pallas as pl


def kernel(x, mu_table, sigma_table):
    raise NotImplementedError("write your pallas kernel here")


# speedup vs baseline: 10.0459x; 10.0459x over previous
import functools

import jax
import jax.numpy as jnp
from jax import lax
from jax.experimental import pallas as pl
from jax.experimental.pallas import tpu as pltpu
from jax.experimental.pallas import tpu_sc as plsc

V_DIM = 100000
D_DIM = 64
BATCH = 4096
HIST_LEN = 50

NC = 2
NS = 16
NW = NC * NS
LANES = 16

N_IDX = BATCH * HIST_LEN
PER_W = N_IDX // NW
CHUNK = 640
N_CHUNKS = PER_W // CHUNK


def _softplus_vec(v):
    a = jnp.abs(v)
    u = jnp.exp(-a)
    t = u / (2.0 + u)
    t2 = t * t
    s = 1.0 + t2 * (1.0 / 3.0 + t2 * (1.0 / 5.0 + t2 * (1.0 / 7.0 + t2 * (1.0 / 9.0))))
    log1p_u = 2.0 * t * s
    return jnp.maximum(v, 0.0) + log1p_u


def _sc_body(x_hbm, mu_t_hbm, sg_t_hbm, mu_out_hbm, sg_out_hbm,
             idx_v, mu_buf, sg_buf, sem_mu, sem_sg):
    c = lax.axis_index("c")
    s = lax.axis_index("s")
    wid = s * NC + c
    base = wid * PER_W
    pltpu.sync_copy(x_hbm.at[pl.ds(base, PER_W)], idx_v)
    for k in range(N_CHUNKS):
        idx_slice = idx_v.at[pl.ds(k * CHUNK, CHUNK)]
        cp_mu = pltpu.async_copy(mu_t_hbm.at[idx_slice], mu_buf, sem_mu)
        cp_sg = pltpu.async_copy(sg_t_hbm.at[idx_slice], sg_buf, sem_sg)
        cp_sg.wait()

        def sp_step(i, _):
            off = i * LANES
            sg_buf[pl.ds(off, LANES)] = _softplus_vec(sg_buf[pl.ds(off, LANES)])
            return _

        lax.fori_loop(0, CHUNK // LANES, sp_step, None)
        pltpu.sync_copy(sg_buf, sg_out_hbm.at[pl.ds(base + k * CHUNK, CHUNK)])
        cp_mu.wait()
        pltpu.sync_copy(mu_buf, mu_out_hbm.at[pl.ds(base + k * CHUNK, CHUNK)])


@functools.partial(jax.jit, static_argnums=())
def _run(x_flat, mu_table, sg_flat):
    mesh = plsc.VectorSubcoreMesh(core_axis_name="c", subcore_axis_name="s")
    f = pl.kernel(
        _sc_body,
        out_type=[
            jax.ShapeDtypeStruct((N_IDX, D_DIM), jnp.float32),
            jax.ShapeDtypeStruct((N_IDX,), jnp.float32),
        ],
        mesh=mesh,
        scratch_types=[
            pltpu.VMEM((PER_W,), jnp.int32),
            pltpu.VMEM((CHUNK, D_DIM), jnp.float32),
            pltpu.VMEM((CHUNK,), jnp.float32),
            pltpu.SemaphoreType.DMA,
            pltpu.SemaphoreType.DMA,
        ],
        compiler_params=pltpu.CompilerParams(use_tc_tiling_on_sc=False),
    )
    return f(x_flat, mu_table, sg_flat)


def kernel(x, mu_table, sigma_table):
    x_flat = x.reshape(N_IDX)
    sg_flat = sigma_table.reshape(V_DIM)
    mu_flat, sg_out = _run(x_flat, mu_table, sg_flat)
    mu = mu_flat.reshape(BATCH, HIST_LEN, D_DIM)
    sigma = sg_out.reshape(BATCH, HIST_LEN, 1)
    return (mu, sigma)
